# baseline (speedup 1.0000x reference)
import jax
import jax.numpy as jnp
from jax.experimental import pallas as pl
from jax.experimental.pallas import tpu as pltpu

_NUM_M = 495
_NUM_D = 383
_OFF = 383
_DIM = 128
_LAYERS = 3
_N_EDGE = 5430
_EB = 5430

_RT2 = (((1,), (1,)), ((), ()))


def _body(m_sim_ref, d_sim_ref, Wm_ref, Wd_ref, A1W_ref,
          A1b_ref, A2W_ref, tm_ref, td_ref,
          out_m_ref, out_d_ref):
    f32 = jnp.float32
    dg = jax.lax.dot_general
    Em = dg(m_sim_ref[...], Wm_ref[...], _RT2, preferred_element_type=f32)
    Ed = dg(d_sim_ref[...], Wd_ref[...], _RT2, preferred_element_type=f32)
    EmT = Em.T
    EdT = Ed.T

    A1m = A1W_ref[:, :_DIM]
    A1d = A1W_ref[:, _DIM:]
    A1b_col = A1b_ref[...].reshape(1, _DIM).T
    A2 = A2W_ref[...]

    Pm = jnp.dot(A1m, jnp.maximum(EmT, 0.0), preferred_element_type=f32)
    Pd = jnp.dot(A1d, jnp.maximum(EdT, 0.0), preferred_element_type=f32) + A1b_col

    bf16 = jnp.bfloat16
    Pm_b = Pm.astype(bf16)
    Pd_b = Pd.astype(bf16)
    S = jnp.zeros((_NUM_M, _NUM_D), f32)
    for start in range(0, _N_EDGE, _EB):
        nb = min(_EB, _N_EDGE - start)
        tm_r = tm_ref[start:start + nb].reshape(1, nb).astype(jnp.int16)
        td_r = td_ref[start:start + nb].reshape(1, nb).astype(jnp.int16)
        cmp_m = tm_r == _OFF + jax.lax.broadcasted_iota(jnp.int16, (_NUM_M, nb), 0)
        cmp_d = td_r == jax.lax.broadcasted_iota(jnp.int16, (_NUM_D, nb), 0)
        oh_mT = cmp_m.astype(bf16)
        oh_dT = cmp_d.astype(bf16)
        hT = jnp.tanh(jnp.dot(Pm_b, oh_mT, preferred_element_type=f32)
                      + jnp.dot(Pd_b, oh_dT, preferred_element_type=f32))
        sc = jnp.dot(A2, hT, preferred_element_type=f32)
        vals = jnp.exp(sc).astype(bf16)
        oh_dT_s = jnp.where(cmp_d, vals, jnp.zeros((), bf16))
        S = S + dg(oh_mT, oh_dT_s, _RT2, preferred_element_type=f32)

    rowsum = jnp.sum(S, axis=1, keepdims=True)
    rm = jnp.where(rowsum > 0.0, 1.0 / rowsum, 0.0)
    ST = S.T
    colsum = jnp.sum(ST, axis=1, keepdims=True)
    rd = jnp.where(colsum > 0.0, 1.0 / colsum, 0.0)

    m_acc = jnp.zeros((_NUM_M, _DIM), f32)
    d_acc = jnp.zeros((_NUM_D, _DIM), f32)
    d_emb = Ed
    for _ in range(_LAYERS):
        m_emb = jnp.tanh(jnp.dot(S, d_emb, preferred_element_type=f32) * rm)
        d_emb = jnp.tanh(jnp.dot(ST, m_emb, preferred_element_type=f32) * rd)
        m_acc = m_acc + m_emb
        d_acc = d_acc + d_emb
    out_m_ref[...] = m_acc
    out_d_ref[...] = d_acc


def kernel(m_sim, d_sim, W_m, W_d, A1_W, A1_b, A2_W, train_mirna, train_disease):
    f32 = jnp.float32
    return pl.pallas_call(
        _body,
        out_shape=(
            jax.ShapeDtypeStruct((_NUM_M, _DIM), f32),
            jax.ShapeDtypeStruct((_NUM_D, _DIM), f32),
        ),
    )(m_sim, d_sim, W_m, W_d, A1_W, A1_b, A2_W, train_mirna, train_disease)


# speedup vs baseline: 1.2080x; 1.2080x over previous
import jax
import jax.numpy as jnp
from jax.experimental import pallas as pl
from jax.experimental.pallas import tpu as pltpu

_NUM_M = 495
_NUM_D = 383
_OFF = 383
_DIM = 128
_LAYERS = 3
_N_EDGE = 5430
_EB = 5430

_RT2 = (((1,), (1,)), ((), ()))


def _body(m_sim_ref, d_sim_ref, Wm_ref, Wd_ref, A1W_ref,
          A1b_ref, A2W_ref, tm_ref, td_ref,
          out_m_ref, out_d_ref):
    f32 = jnp.float32
    dg = jax.lax.dot_general
    Em = dg(m_sim_ref[...], Wm_ref[...], _RT2, preferred_element_type=f32)
    Ed = dg(d_sim_ref[...], Wd_ref[...], _RT2, preferred_element_type=f32)
    EmT = Em.T
    EdT = Ed.T

    A1m = A1W_ref[:, :_DIM]
    A1d = A1W_ref[:, _DIM:]
    A1b_col = A1b_ref[...].reshape(1, _DIM).T
    A2 = A2W_ref[...]

    Pm = jnp.dot(A1m, jnp.maximum(EmT, 0.0), preferred_element_type=f32)
    Pd = jnp.dot(A1d, jnp.maximum(EdT, 0.0), preferred_element_type=f32) + A1b_col

    S = jnp.zeros((_NUM_M, _NUM_D), f32)
    for start in range(0, _N_EDGE, _EB):
        nb = min(_EB, _N_EDGE - start)
        tm_r = tm_ref[start:start + nb].reshape(1, nb)
        td_r = td_ref[start:start + nb].reshape(1, nb)
        cmp_m = tm_r == _OFF + jax.lax.broadcasted_iota(jnp.int32, (_NUM_M, nb), 0)
        cmp_d = td_r == jax.lax.broadcasted_iota(jnp.int32, (_NUM_D, nb), 0)
        oh_mT = cmp_m.astype(f32)
        oh_dT = cmp_d.astype(f32)
        hT = jnp.tanh(jnp.dot(Pm, oh_mT, preferred_element_type=f32)
                      + jnp.dot(Pd, oh_dT, preferred_element_type=f32))
        sc = jnp.dot(A2, hT, preferred_element_type=f32)
        vals = jnp.exp(sc)
        oh_dT_s = jnp.where(cmp_d, vals, 0.0)
        S = S + dg(oh_mT, oh_dT_s, _RT2, preferred_element_type=f32)

    rowsum = jnp.sum(S, axis=1, keepdims=True)
    rm = jnp.where(rowsum > 0.0, 1.0 / rowsum, 0.0)
    ST = S.T
    colsum = jnp.sum(ST, axis=1, keepdims=True)
    rd = jnp.where(colsum > 0.0, 1.0 / colsum, 0.0)

    m_acc = jnp.zeros((_NUM_M, _DIM), f32)
    d_acc = jnp.zeros((_NUM_D, _DIM), f32)
    d_emb = Ed
    for _ in range(_LAYERS):
        m_emb = jnp.tanh(jnp.dot(S, d_emb, preferred_element_type=f32) * rm)
        d_emb = jnp.tanh(jnp.dot(ST, m_emb, preferred_element_type=f32) * rd)
        m_acc = m_acc + m_emb
        d_acc = d_acc + d_emb
    out_m_ref[...] = m_acc
    out_d_ref[...] = d_acc


def kernel(m_sim, d_sim, W_m, W_d, A1_W, A1_b, A2_W, train_mirna, train_disease):
    f32 = jnp.float32
    return pl.pallas_call(
        _body,
        out_shape=(
            jax.ShapeDtypeStruct((_NUM_M, _DIM), f32),
            jax.ShapeDtypeStruct((_NUM_D, _DIM), f32),
        ),
    )(m_sim, d_sim, W_m, W_d, A1_W, A1_b, A2_W, train_mirna, train_disease)
